# baseline (speedup 1.0000x reference)
import jax
import jax.numpy as jnp
from jax import lax
from jax.experimental import pallas as pl
from jax.experimental.pallas import tpu as pltpu
from jax.experimental.pallas import tpu_sc as plsc

_MARGIN = 1.0
_BIG = float("inf")
_B = 256
_D = 64
_NC = 2
_NS = 16
_L = 16
_NW = _NC * _NS
_APW = _B // _NW
_NCH = _B // _L
_PST = _B + _L


def _pdist_body(x_ref, out_ref):
    x = x_ref[...]
    xx = x * x
    n2_col = jnp.sum(xx, axis=1, keepdims=True)
    ones = jnp.ones((1, _D), jnp.float32)
    n2_row = lax.dot_general(ones, xx, (((1,), (1,)), ((), ())),
                             preferred_element_type=jnp.float32)
    gram = lax.dot_general(x, x, (((1,), (1,)), ((), ())),
                           preferred_element_type=jnp.float32)
    out_ref[...] = n2_col + n2_row - 2.0 * gram


_pdist = pl.pallas_call(
    _pdist_body,
    out_shape=jax.ShapeDtypeStruct((_B, _B), jnp.float32),
)


def _sc_body(d_hbm, lbl_hbm, out_hbm,
             lbl_v, drow_v, negw_v, posb_v, stage_v, sem):
    cid = lax.axis_index("c")
    sid = lax.axis_index("s")
    wid = sid * _NC + cid
    base = wid * _APW

    c1 = pltpu.async_copy(lbl_hbm, lbl_v.at[pl.ds(0, _B)], sem)
    c2 = pltpu.async_copy(d_hbm.at[pl.ds(base, _APW)], drow_v, sem)
    c1.wait()
    c2.wait()

    acc = jnp.zeros((_L,), jnp.float32)
    cnt = jnp.float32(0.0)

    liv = [None] * _APW
    gv = [None] * _APW
    for a in range(_APW):
        g = base + a
        li = lbl_v[pl.ds(g, _L)][0]
        liv[a] = jnp.full((_L,), li, jnp.int32)
        gv[a] = jnp.full((_L,), g, jnp.int32)

    poff = [jnp.int32(0)] * _APW
    nsum = [jnp.int32(0)] * _APW
    for c in range(_NCH):
        lbl = lbl_v[pl.ds(c * _L, _L)]
        jidx = lax.iota(jnp.int32, _L) + (c * _L)
        for a in range(_APW):
            dch = drow_v[a, pl.ds(c * _L, _L)]
            same = lbl == liv[a]
            pos_m = same & (jidx > gv[a])
            neg_m = jnp.logical_not(same)
            negw_v[a, pl.ds(c * _L, _L)] = jnp.where(same, _BIG, dch)
            plsc.store_compressed(
                posb_v.at[pl.ds(a * _PST + poff[a], _L)],
                dch + _MARGIN, mask=pos_m)
            poff[a] = poff[a] + plsc.all_reduce_population_count(pos_m)[0]
            nsum[a] = nsum[a] + plsc.all_reduce_population_count(neg_m)[0]

    for a in range(_APW):
        cnt = cnt + (poff[a] * nsum[a]).astype(jnp.float32)

        def _pos_body(p, acc_, a=a):
            tv = plsc.load_gather(
                posb_v, (jnp.full((_L,), a * _PST, jnp.int32) + p,))
            vals = [jnp.maximum(tv - negw_v[a, pl.ds(c * _L, _L)], 0.0)
                    for c in range(_NCH)]
            while len(vals) > 1:
                vals = [vals[i] + vals[i + 1]
                        for i in range(0, len(vals), 2)]
            return acc_ + vals[0]

        acc = plsc.parallel_loop(0, poff[a], unroll=2,
                                 carry=acc)(_pos_body)

    stage_v[pl.ds(0, _L)] = acc
    stage_v[pl.ds(_L, _L)] = jnp.full((_L,), cnt, jnp.float32)
    pltpu.sync_copy(stage_v, out_hbm.at[wid])


_sc_reduce_cache = []


def _sc_reduce():
    if not _sc_reduce_cache:
        _sc_reduce_cache.append(pl.kernel(
            _sc_body,
            out_type=jax.ShapeDtypeStruct((_NW, 2 * _L), jnp.float32),
            mesh=plsc.VectorSubcoreMesh(core_axis_name="c",
                                        subcore_axis_name="s",
                                        num_cores=_NC, num_subcores=_NS),
            compiler_params=pltpu.CompilerParams(needs_layout_passes=False),
            scratch_types=[
                pltpu.VMEM((_B + _L,), jnp.int32),
                pltpu.VMEM((_APW, _B), jnp.float32),
                pltpu.VMEM((_APW, _B), jnp.float32),
                pltpu.VMEM((_APW * _PST,), jnp.float32),
                pltpu.VMEM((2 * _L,), jnp.float32),
                pltpu.SemaphoreType.DMA,
            ],
        ))
    return _sc_reduce_cache[0]


def kernel(embeddings, target):
    dmat = _pdist(embeddings)
    out = _sc_reduce()(dmat, target.astype(jnp.int32))
    return jnp.sum(out[:, :_L]) / jnp.sum(out[:, _L])


# speedup vs baseline: 1.0916x; 1.0916x over previous
import jax
import jax.numpy as jnp
from jax import lax
from jax.experimental import pallas as pl
from jax.experimental.pallas import tpu as pltpu
from jax.experimental.pallas import tpu_sc as plsc

_MARGIN = 1.0
_BIG = float("inf")
_B = 256
_D = 64
_NC = 2
_NS = 16
_L = 16
_NW = _NC * _NS
_APW = _B // _NW
_NCH = _B // _L
_PST = _B + _L


def _pdist_body(x_ref, out_ref):
    x = x_ref[...]
    xx = x * x
    n2_col = jnp.sum(xx, axis=1, keepdims=True)
    ones = jnp.ones((1, _D), jnp.float32)
    n2_row = lax.dot_general(ones, xx, (((1,), (1,)), ((), ())),
                             preferred_element_type=jnp.float32)
    gram = lax.dot_general(x, x, (((1,), (1,)), ((), ())),
                           preferred_element_type=jnp.float32)
    out_ref[...] = n2_col + n2_row - 2.0 * gram


_pdist = pl.pallas_call(
    _pdist_body,
    out_shape=jax.ShapeDtypeStruct((_B, _B), jnp.float32),
)


def _sc_body(d_hbm, lbl_hbm, out_hbm,
             lbl_v, drow_v, negw_v, posb_v, stage_v, sem):
    cid = lax.axis_index("c")
    sid = lax.axis_index("s")
    wid = sid * _NC + cid
    base = wid * _APW

    c1 = pltpu.async_copy(lbl_hbm, lbl_v.at[pl.ds(0, _B)], sem)
    c2 = pltpu.async_copy(d_hbm.at[pl.ds(base, _APW)], drow_v, sem)
    c1.wait()
    c2.wait()

    acc = jnp.zeros((_L,), jnp.float32)
    cnt = jnp.float32(0.0)

    liv = [None] * _APW
    gv = [None] * _APW
    for a in range(_APW):
        g = base + a
        li = lbl_v[pl.ds(g, _L)][0]
        liv[a] = jnp.full((_L,), li, jnp.int32)
        gv[a] = jnp.full((_L,), g, jnp.int32)

    poff = [jnp.int32(0)] * _APW
    nsum = [jnp.int32(0)] * _APW
    for c in range(_NCH):
        lbl = lbl_v[pl.ds(c * _L, _L)]
        jidx = lax.iota(jnp.int32, _L) + (c * _L)
        for a in range(_APW):
            dch = drow_v[a, pl.ds(c * _L, _L)]
            same = lbl == liv[a]
            pos_m = same & (jidx > gv[a])
            neg_m = jnp.logical_not(same)
            negw_v[a, pl.ds(c * _L, _L)] = jnp.where(same, _BIG, dch)
            plsc.store_compressed(
                posb_v.at[pl.ds(a * _PST + poff[a], _L)],
                dch + _MARGIN, mask=pos_m)
            poff[a] = poff[a] + plsc.all_reduce_population_count(pos_m)[0]
            nsum[a] = nsum[a] + plsc.all_reduce_population_count(neg_m)[0]

    for a in range(_APW):
        cnt = cnt + (poff[a] * nsum[a]).astype(jnp.float32)

        def _pos_body(p, acc_, a=a):
            t = posb_v[pl.ds(a * _PST + p, _L)][0]
            tv = jnp.full((_L,), t, jnp.float32)
            vals = [jnp.maximum(tv - negw_v[a, pl.ds(c * _L, _L)], 0.0)
                    for c in range(_NCH)]
            while len(vals) > 1:
                vals = [vals[i] + vals[i + 1]
                        for i in range(0, len(vals), 2)]
            return acc_ + vals[0]

        acc = lax.fori_loop(0, poff[a], _pos_body, acc)

    stage_v[pl.ds(0, _L)] = acc
    stage_v[pl.ds(_L, _L)] = jnp.full((_L,), cnt, jnp.float32)
    pltpu.sync_copy(stage_v, out_hbm.at[wid])


_sc_reduce_cache = []


def _sc_reduce():
    if not _sc_reduce_cache:
        _sc_reduce_cache.append(pl.kernel(
            _sc_body,
            out_type=jax.ShapeDtypeStruct((_NW, 2 * _L), jnp.float32),
            mesh=plsc.VectorSubcoreMesh(core_axis_name="c",
                                        subcore_axis_name="s",
                                        num_cores=_NC, num_subcores=_NS),
            compiler_params=pltpu.CompilerParams(needs_layout_passes=False),
            scratch_types=[
                pltpu.VMEM((_B + _L,), jnp.int32),
                pltpu.VMEM((_APW, _B), jnp.float32),
                pltpu.VMEM((_APW, _B), jnp.float32),
                pltpu.VMEM((_APW * _PST,), jnp.float32),
                pltpu.VMEM((2 * _L,), jnp.float32),
                pltpu.SemaphoreType.DMA,
            ],
        ))
    return _sc_reduce_cache[0]


def kernel(embeddings, target):
    dmat = _pdist(embeddings)
    out = _sc_reduce()(dmat, target.astype(jnp.int32))
    return jnp.sum(out[:, :_L]) / jnp.sum(out[:, _L])
